# baseline (speedup 1.0000x reference)
import functools

import jax
import jax.numpy as jnp
from jax import lax
from jax.experimental import pallas as pl
from jax.experimental.pallas import tpu as pltpu
from jax.experimental.pallas import tpu_sc as plsc

_DIMS = (119, 5, 12, 12, 10, 6, 6, 2, 2)
_HID = 256
_GROUPS = ((0,), (1, 2), (3, 4), (5, 6), (7, 8))
_L = 16
_NC = 2
_NS = 16
_NW = _NC * _NS
_CB = 128
_NF = len(_DIMS)


def _group_layout():
    sizes = []
    for grp in _GROUPS:
        r = 1
        for j in grp:
            r *= _DIMS[j]
        sizes.append(r)
    offs, acc = [], 0
    for s in sizes:
        offs.append(acc)
        acc += s
    return tuple(sizes), tuple(offs), acc


_GSIZES, _GOFFS, _TROWS = _group_layout()


def _build_table(tables):
    parts = []
    for grp in _GROUPS:
        t = tables[grp[0]]
        for j in grp[1:]:
            t = (t[:, None, :] + tables[j][None, :, :]).reshape(-1, _HID)
        parts.append(t)
    tb = jnp.concatenate(parts, 0).astype(jnp.bfloat16)
    lo = lax.bitcast_convert_type(tb[:, :_HID // 2], jnp.uint16)
    hi = lax.bitcast_convert_type(tb[:, _HID // 2:], jnp.uint16)
    packed = (hi.astype(jnp.uint32) << 16) | lo.astype(jnp.uint32)
    return lax.bitcast_convert_type(packed, jnp.int32).reshape(-1)


def _splat(vec, lane):
    idx = jnp.full((_L, 1), lane, jnp.int32)
    return lax.gather(
        vec, idx,
        dimension_numbers=lax.GatherDimensionNumbers(
            offset_dims=(), collapsed_slice_dims=(0,), start_index_map=(0,)),
        slice_sizes=(1,),
        mode=lax.GatherScatterMode.PROMISE_IN_BOUNDS)


@functools.lru_cache(maxsize=None)
def _make(n_nodes):
    n_chunks = -(-n_nodes // _CB)
    kmax = -(-n_chunks // _NW)
    last_base = n_nodes - _CB

    mesh = plsc.VectorSubcoreMesh(
        core_axis_name="c", subcore_axis_name="s",
        num_cores=_NC, num_subcores=_NS)

    @functools.partial(
        pl.kernel,
        out_type=jax.ShapeDtypeStruct((n_nodes, _HID), jnp.float32),
        mesh=mesh,
        scratch_types=[
            pltpu.VMEM((_TROWS * _HID // 2,), jnp.int32),
            pltpu.VMEM((2 * _NF * _CB,), jnp.int32),
            pltpu.VMEM((2 * _CB, _HID), jnp.float32),
            pltpu.SemaphoreType.DMA,
            pltpu.SemaphoreType.DMA,
            pltpu.SemaphoreType.DMA,
            pltpu.SemaphoreType.DMA,
        ],
        compiler_params=pltpu.CompilerParams(needs_layout_passes=False),
    )
    def sc_kernel(x_f, tflat, out, tbl_v, xv, out_v,
                  in_sem0, in_sem1, out_sem0, out_sem1):
        in_sems = (in_sem0, in_sem1)
        out_sems = (out_sem0, out_sem1)
        wid = lax.axis_index("s") * _NC + lax.axis_index("c")
        pltpu.sync_copy(tflat, tbl_v)
        iota = lax.broadcasted_iota(jnp.int32, (_L,), 0)

        def chunk_base(kk):
            cid = jnp.minimum(wid + _NW * kk, n_chunks - 1)
            return jnp.minimum(cid * _CB, last_base)

        def in_copies(kk, buf):
            base = chunk_base(kk)
            return [pltpu.make_async_copy(
                        x_f.at[pl.ds(j * n_nodes + base, _CB)],
                        xv.at[pl.ds((buf * _NF + j) * _CB, _CB)],
                        in_sems[buf])
                    for j in range(_NF)]

        def start_in(kk, buf):
            for c in in_copies(kk, buf):
                c.start()

        start_in(0, 0)

        def do_chunk(kk, buf):
            @pl.when(kk + 1 < kmax)
            def _():
                start_in(kk + 1, 1 - buf)

            for c in in_copies(kk, buf):
                c.wait()

            @pl.when(kk >= 2)
            def _():
                pltpu.make_async_copy(
                    out_v.at[pl.ds(buf * _CB, _CB)],
                    out.at[pl.ds(chunk_base(kk), _CB)],
                    out_sems[buf]).wait()

            for g in range(_CB // _L):
                xg = [xv[pl.ds((buf * _NF + j) * _CB + g * _L, _L)]
                      for j in range(_NF)]
                rb = []
                for gi, grp in enumerate(_GROUPS):
                    idx = xg[grp[0]]
                    for j in grp[1:]:
                        idx = idx * _DIMS[j] + xg[j]
                    rb.append((idx + _GOFFS[gi]) * (_HID // 2))

                @plsc.parallel_loop(0, _L)
                def _node(n, rb=rb, g=g, buf=buf):
                    pre = [_splat(r, n) + iota for r in rb]
                    rowv = jnp.full((_L,), buf * _CB + g * _L + n, jnp.int32)

                    @plsc.parallel_loop(0, _HID // 2, _L, unroll=8)
                    def _cols(col, pre=pre, rowv=rowv):
                        acc = None
                        for p in pre:
                            w = plsc.bitcast(
                                plsc.load_gather(tbl_v, [p + col]),
                                jnp.bfloat16)
                            acc = w if acc is None else acc + w
                        acc_lo, acc_hi = plsc.unpack(
                            acc, format=plsc.PackFormat.INTERLEAVED,
                            preferred_element_type=jnp.float32)
                        plsc.store_scatter(
                            out_v, [rowv, iota + col], acc_lo)
                        plsc.store_scatter(
                            out_v, [rowv, iota + (col + _HID // 2)],
                            acc_hi)

            pltpu.make_async_copy(
                out_v.at[pl.ds(buf * _CB, _CB)],
                out.at[pl.ds(chunk_base(kk), _CB)],
                out_sems[buf]).start()

        @pl.loop(0, (kmax + 1) // 2)
        def _pair(kk2):
            kk = kk2 * 2
            do_chunk(kk, 0)

            @pl.when(kk + 1 < kmax)
            def _():
                do_chunk(kk + 1, 1)

        for buf in range(2):
            pltpu.make_async_copy(
                out_v.at[pl.ds(buf * _CB, _CB)], out.at[pl.ds(0, _CB)],
                out_sems[buf]).wait()

    return sc_kernel


def kernel(x, tables):
    tflat = _build_table(tables)
    x_f = x.T.reshape(-1)
    return _make(x.shape[0])(x_f, tflat)


# speedup vs baseline: 1.3459x; 1.3459x over previous
import functools

import jax
import jax.numpy as jnp
from jax import lax
from jax.experimental import pallas as pl
from jax.experimental.pallas import tpu as pltpu
from jax.experimental.pallas import tpu_sc as plsc

_DIMS = (119, 5, 12, 12, 10, 6, 6, 2, 2)
_HID = 256
_GROUPS = ((0,), (1, 2), (3, 4), (5, 6), (7, 8))
_L = 16
_NC = 2
_NS = 16
_NW = _NC * _NS
_CB = 128
_NF = len(_DIMS)


def _group_layout():
    sizes = []
    for grp in _GROUPS:
        r = 1
        for j in grp:
            r *= _DIMS[j]
        sizes.append(r)
    offs, acc = [], 0
    for s in sizes:
        offs.append(acc)
        acc += s
    return tuple(sizes), tuple(offs), acc


_GSIZES, _GOFFS, _TROWS = _group_layout()


def _build_table(tables):
    parts = []
    for grp in _GROUPS:
        t = tables[grp[0]]
        for j in grp[1:]:
            t = (t[:, None, :] + tables[j][None, :, :]).reshape(-1, _HID)
        parts.append(t)
    tb = jnp.concatenate(parts, 0).astype(jnp.bfloat16)
    lo = lax.bitcast_convert_type(tb[:, :_HID // 2], jnp.uint16)
    hi = lax.bitcast_convert_type(tb[:, _HID // 2:], jnp.uint16)
    packed = (hi.astype(jnp.uint32) << 16) | lo.astype(jnp.uint32)
    return lax.bitcast_convert_type(packed, jnp.int32).reshape(-1)


def _splat(vec, lane):
    idx = jnp.full((_L, 1), lane, jnp.int32)
    return lax.gather(
        vec, idx,
        dimension_numbers=lax.GatherDimensionNumbers(
            offset_dims=(), collapsed_slice_dims=(0,), start_index_map=(0,)),
        slice_sizes=(1,),
        mode=lax.GatherScatterMode.PROMISE_IN_BOUNDS)


@functools.lru_cache(maxsize=None)
def _make(n_nodes):
    n_chunks = -(-n_nodes // _CB)
    kmax = -(-n_chunks // _NW)
    last_base = n_nodes - _CB

    mesh = plsc.VectorSubcoreMesh(
        core_axis_name="c", subcore_axis_name="s",
        num_cores=_NC, num_subcores=_NS)

    @functools.partial(
        pl.kernel,
        out_type=jax.ShapeDtypeStruct((n_nodes, _HID), jnp.float32),
        mesh=mesh,
        scratch_types=[
            pltpu.VMEM((_TROWS * _HID // 2,), jnp.int32),
            pltpu.VMEM((2 * _NF * _CB,), jnp.int32),
            pltpu.VMEM((2 * _CB, _HID), jnp.float32),
            pltpu.SemaphoreType.DMA,
            pltpu.SemaphoreType.DMA,
            pltpu.SemaphoreType.DMA,
            pltpu.SemaphoreType.DMA,
        ],
        compiler_params=pltpu.CompilerParams(needs_layout_passes=False),
    )
    def sc_kernel(x_f, tflat, out, tbl_v, xv, out_v,
                  in_sem0, in_sem1, out_sem0, out_sem1):
        in_sems = (in_sem0, in_sem1)
        out_sems = (out_sem0, out_sem1)
        wid = lax.axis_index("s") * _NC + lax.axis_index("c")
        pltpu.sync_copy(tflat, tbl_v)
        iota = lax.broadcasted_iota(jnp.int32, (_L,), 0)

        def chunk_base(kk):
            cid = jnp.minimum(wid + _NW * kk, n_chunks - 1)
            return jnp.minimum(cid * _CB, last_base)

        def in_copies(kk, buf):
            base = chunk_base(kk)
            return [pltpu.make_async_copy(
                        x_f.at[pl.ds(j * n_nodes + base, _CB)],
                        xv.at[pl.ds((buf * _NF + j) * _CB, _CB)],
                        in_sems[buf])
                    for j in range(_NF)]

        def start_in(kk, buf):
            for c in in_copies(kk, buf):
                c.start()

        start_in(0, 0)

        def do_chunk(kk, buf):
            @pl.when(kk + 1 < kmax)
            def _():
                start_in(kk + 1, 1 - buf)

            for c in in_copies(kk, buf):
                c.wait()

            @pl.when(kk >= 2)
            def _():
                pltpu.make_async_copy(
                    out_v.at[pl.ds(buf * _CB, _CB)],
                    out.at[pl.ds(chunk_base(kk), _CB)],
                    out_sems[buf]).wait()

            for g in range(_CB // _L):
                xg = [xv[pl.ds((buf * _NF + j) * _CB + g * _L, _L)]
                      for j in range(_NF)]
                rb = []
                for gi, grp in enumerate(_GROUPS):
                    idx = xg[grp[0]]
                    for j in grp[1:]:
                        idx = idx * _DIMS[j] + xg[j]
                    rb.append((idx + _GOFFS[gi]) * (_HID // 2))

                @pl.loop(0, _L)
                def _node(n, rb=rb, g=g, buf=buf):
                    pre = [_splat(r, n) + iota for r in rb]
                    rowv = jnp.full((_L,), buf * _CB + g * _L + n, jnp.int32)

                    @plsc.parallel_loop(0, _HID // 2, _L, unroll=4)
                    def _cols(col, pre=pre, rowv=rowv):
                        acc = None
                        for p in pre:
                            w = plsc.bitcast(
                                plsc.load_gather(tbl_v, [p + col]),
                                jnp.bfloat16)
                            acc = w if acc is None else acc + w
                        acc_lo, acc_hi = plsc.unpack(
                            acc, format=plsc.PackFormat.INTERLEAVED,
                            preferred_element_type=jnp.float32)
                        plsc.store_scatter(
                            out_v, [rowv, iota + col], acc_lo)
                        plsc.store_scatter(
                            out_v, [rowv, iota + (col + _HID // 2)],
                            acc_hi)

            pltpu.make_async_copy(
                out_v.at[pl.ds(buf * _CB, _CB)],
                out.at[pl.ds(chunk_base(kk), _CB)],
                out_sems[buf]).start()

        @pl.loop(0, (kmax + 1) // 2)
        def _pair(kk2):
            kk = kk2 * 2
            do_chunk(kk, 0)

            @pl.when(kk + 1 < kmax)
            def _():
                do_chunk(kk + 1, 1)

        for buf in range(2):
            pltpu.make_async_copy(
                out_v.at[pl.ds(buf * _CB, _CB)], out.at[pl.ds(0, _CB)],
                out_sems[buf]).wait()

    return sc_kernel


def kernel(x, tables):
    tflat = _build_table(tables)
    x_f = x.T.reshape(-1)
    return _make(x.shape[0])(x_f, tflat)


# speedup vs baseline: 1.4311x; 1.0633x over previous
import functools

import jax
import jax.numpy as jnp
from jax import lax
from jax.experimental import pallas as pl
from jax.experimental.pallas import tpu as pltpu
from jax.experimental.pallas import tpu_sc as plsc

_DIMS = (119, 5, 12, 12, 10, 6, 6, 2, 2)
_HID = 256
_GROUPS = ((0,), (1, 2), (3, 4), (5, 6, 7, 8))
_L = 16
_NC = 2
_NS = 16
_NW = _NC * _NS
_CB = 128
_NF = len(_DIMS)


def _group_layout():
    sizes = []
    for grp in _GROUPS:
        r = 1
        for j in grp:
            r *= _DIMS[j]
        sizes.append(r)
    offs, acc = [], 0
    for s in sizes:
        offs.append(acc)
        acc += s
    return tuple(sizes), tuple(offs), acc


_GSIZES, _GOFFS, _TROWS = _group_layout()


def _build_table(tables):
    parts = []
    for grp in _GROUPS:
        t = tables[grp[0]]
        for j in grp[1:]:
            t = (t[:, None, :] + tables[j][None, :, :]).reshape(-1, _HID)
        parts.append(t)
    tb = jnp.concatenate(parts, 0).astype(jnp.bfloat16)
    lo = lax.bitcast_convert_type(tb[:, :_HID // 2], jnp.uint16)
    hi = lax.bitcast_convert_type(tb[:, _HID // 2:], jnp.uint16)
    packed = (hi.astype(jnp.uint32) << 16) | lo.astype(jnp.uint32)
    return lax.bitcast_convert_type(packed, jnp.int32).reshape(-1)


def _splat(vec, lane):
    idx = jnp.full((_L, 1), lane, jnp.int32)
    return lax.gather(
        vec, idx,
        dimension_numbers=lax.GatherDimensionNumbers(
            offset_dims=(), collapsed_slice_dims=(0,), start_index_map=(0,)),
        slice_sizes=(1,),
        mode=lax.GatherScatterMode.PROMISE_IN_BOUNDS)


@functools.lru_cache(maxsize=None)
def _make(n_nodes):
    n_chunks = -(-n_nodes // _CB)
    kmax = -(-n_chunks // _NW)
    last_base = n_nodes - _CB

    mesh = plsc.VectorSubcoreMesh(
        core_axis_name="c", subcore_axis_name="s",
        num_cores=_NC, num_subcores=_NS)

    @functools.partial(
        pl.kernel,
        out_type=jax.ShapeDtypeStruct((n_nodes, _HID), jnp.float32),
        mesh=mesh,
        scratch_types=[
            pltpu.VMEM((_TROWS * _HID // 2,), jnp.int32),
            pltpu.VMEM((2 * _NF * _CB,), jnp.int32),
            pltpu.VMEM((2 * _CB, _HID), jnp.float32),
            pltpu.SemaphoreType.DMA,
            pltpu.SemaphoreType.DMA,
            pltpu.SemaphoreType.DMA,
            pltpu.SemaphoreType.DMA,
        ],
        compiler_params=pltpu.CompilerParams(needs_layout_passes=False),
    )
    def sc_kernel(x_f, tflat, out, tbl_v, xv, out_v,
                  in_sem0, in_sem1, out_sem0, out_sem1):
        in_sems = (in_sem0, in_sem1)
        out_sems = (out_sem0, out_sem1)
        wid = lax.axis_index("s") * _NC + lax.axis_index("c")
        pltpu.sync_copy(tflat, tbl_v)
        iota = lax.broadcasted_iota(jnp.int32, (_L,), 0)

        def chunk_base(kk):
            cid = jnp.minimum(wid + _NW * kk, n_chunks - 1)
            return jnp.minimum(cid * _CB, last_base)

        def in_copies(kk, buf):
            base = chunk_base(kk)
            return [pltpu.make_async_copy(
                        x_f.at[pl.ds(j * n_nodes + base, _CB)],
                        xv.at[pl.ds((buf * _NF + j) * _CB, _CB)],
                        in_sems[buf])
                    for j in range(_NF)]

        def start_in(kk, buf):
            for c in in_copies(kk, buf):
                c.start()

        start_in(0, 0)

        def do_chunk(kk, buf):
            @pl.when(kk + 1 < kmax)
            def _():
                start_in(kk + 1, 1 - buf)

            for c in in_copies(kk, buf):
                c.wait()

            @pl.when(kk >= 2)
            def _():
                pltpu.make_async_copy(
                    out_v.at[pl.ds(buf * _CB, _CB)],
                    out.at[pl.ds(chunk_base(kk), _CB)],
                    out_sems[buf]).wait()

            for g in range(_CB // _L):
                xg = [xv[pl.ds((buf * _NF + j) * _CB + g * _L, _L)]
                      for j in range(_NF)]
                rb = []
                for gi, grp in enumerate(_GROUPS):
                    idx = xg[grp[0]]
                    for j in grp[1:]:
                        idx = idx * _DIMS[j] + xg[j]
                    rb.append((idx + _GOFFS[gi]) * (_HID // 2))

                @pl.loop(0, _L)
                def _node(n, rb=rb, g=g, buf=buf):
                    pre = [_splat(r, n) + iota for r in rb]
                    rowv = jnp.full((_L,), buf * _CB + g * _L + n, jnp.int32)

                    @plsc.parallel_loop(0, _HID // 2, _L, unroll=4)
                    def _cols(col, pre=pre, rowv=rowv):
                        acc = None
                        for p in pre:
                            w = plsc.bitcast(
                                plsc.load_gather(tbl_v, [p + col]),
                                jnp.bfloat16)
                            acc = w if acc is None else acc + w
                        acc_lo, acc_hi = plsc.unpack(
                            acc, format=plsc.PackFormat.INTERLEAVED,
                            preferred_element_type=jnp.float32)
                        plsc.store_scatter(
                            out_v, [rowv, iota + col], acc_lo)
                        plsc.store_scatter(
                            out_v, [rowv, iota + (col + _HID // 2)],
                            acc_hi)

            pltpu.make_async_copy(
                out_v.at[pl.ds(buf * _CB, _CB)],
                out.at[pl.ds(chunk_base(kk), _CB)],
                out_sems[buf]).start()

        @pl.loop(0, (kmax + 1) // 2)
        def _pair(kk2):
            kk = kk2 * 2
            do_chunk(kk, 0)

            @pl.when(kk + 1 < kmax)
            def _():
                do_chunk(kk + 1, 1)

        for buf in range(2):
            pltpu.make_async_copy(
                out_v.at[pl.ds(buf * _CB, _CB)], out.at[pl.ds(0, _CB)],
                out_sems[buf]).wait()

    return sc_kernel


def kernel(x, tables):
    tflat = _build_table(tables)
    x_f = x.T.reshape(-1)
    return _make(x.shape[0])(x_f, tflat)


# speedup vs baseline: 1.6623x; 1.1616x over previous
import functools

import jax
import jax.numpy as jnp
from jax import lax
from jax.experimental import pallas as pl
from jax.experimental.pallas import tpu as pltpu
from jax.experimental.pallas import tpu_sc as plsc

_DIMS = (119, 5, 12, 12, 10, 6, 6, 2, 2)
_HID = 256
_GROUPS = ((0,), (1, 2), (3, 4), (5, 6, 7, 8))
_L = 16
_NC = 2
_NS = 16
_NW = _NC * _NS
_CB = 128
_NF = len(_DIMS)


def _group_layout():
    sizes = []
    for grp in _GROUPS:
        r = 1
        for j in grp:
            r *= _DIMS[j]
        sizes.append(r)
    offs, acc = [], 0
    for s in sizes:
        offs.append(acc)
        acc += s
    return tuple(sizes), tuple(offs), acc


_GSIZES, _GOFFS, _TROWS = _group_layout()


def _build_table(tables):
    parts = []
    for grp in _GROUPS:
        t = tables[grp[0]]
        for j in grp[1:]:
            t = (t[:, None, :] + tables[j][None, :, :]).reshape(-1, _HID)
        parts.append(t)
    tb = jnp.concatenate(parts, 0).astype(jnp.bfloat16)
    lo = lax.bitcast_convert_type(tb[:, :_HID // 2], jnp.uint16)
    hi = lax.bitcast_convert_type(tb[:, _HID // 2:], jnp.uint16)
    packed = (hi.astype(jnp.uint32) << 16) | lo.astype(jnp.uint32)
    return lax.bitcast_convert_type(packed, jnp.int32).reshape(-1)


def _splat(vec, lane):
    idx = jnp.full((_L, 1), lane, jnp.int32)
    return lax.gather(
        vec, idx,
        dimension_numbers=lax.GatherDimensionNumbers(
            offset_dims=(), collapsed_slice_dims=(0,), start_index_map=(0,)),
        slice_sizes=(1,),
        mode=lax.GatherScatterMode.PROMISE_IN_BOUNDS)


@functools.lru_cache(maxsize=None)
def _make(n_nodes):
    n_chunks = -(-n_nodes // _CB)
    kmax = -(-n_chunks // _NW)
    last_base = n_nodes - _CB

    mesh = plsc.VectorSubcoreMesh(
        core_axis_name="c", subcore_axis_name="s",
        num_cores=_NC, num_subcores=_NS)

    @functools.partial(
        pl.kernel,
        out_type=jax.ShapeDtypeStruct((n_nodes, _HID), jnp.float32),
        mesh=mesh,
        scratch_types=[
            pltpu.VMEM((_TROWS * _HID // 2,), jnp.int32),
            pltpu.VMEM((2 * _NF * _CB,), jnp.int32),
            pltpu.VMEM((2 * _CB, _HID), jnp.float32),
            pltpu.SemaphoreType.DMA,
            pltpu.SemaphoreType.DMA,
            pltpu.SemaphoreType.DMA,
            pltpu.SemaphoreType.DMA,
        ],
        compiler_params=pltpu.CompilerParams(needs_layout_passes=False),
    )
    def sc_kernel(x_f, tflat, out, tbl_v, xv, out_v,
                  in_sem0, in_sem1, out_sem0, out_sem1):
        in_sems = (in_sem0, in_sem1)
        out_sems = (out_sem0, out_sem1)
        wid = lax.axis_index("s") * _NC + lax.axis_index("c")
        pltpu.sync_copy(tflat, tbl_v)
        iota = lax.broadcasted_iota(jnp.int32, (_L,), 0)

        def chunk_base(kk):
            cid = jnp.minimum(wid + _NW * kk, n_chunks - 1)
            return jnp.minimum(cid * _CB, last_base)

        def in_copies(kk, buf):
            base = chunk_base(kk)
            return [pltpu.make_async_copy(
                        x_f.at[pl.ds(j * n_nodes + base, _CB)],
                        xv.at[pl.ds((buf * _NF + j) * _CB, _CB)],
                        in_sems[buf])
                    for j in range(_NF)]

        def start_in(kk, buf):
            for c in in_copies(kk, buf):
                c.start()

        start_in(0, 0)

        def do_chunk(kk, buf):
            @pl.when(kk + 1 < kmax)
            def _():
                start_in(kk + 1, 1 - buf)

            for c in in_copies(kk, buf):
                c.wait()

            @pl.when(kk >= 2)
            def _():
                pltpu.make_async_copy(
                    out_v.at[pl.ds(buf * _CB, _CB)],
                    out.at[pl.ds(chunk_base(kk), _CB)],
                    out_sems[buf]).wait()

            for g in range(_CB // _L):
                xg = [xv[pl.ds((buf * _NF + j) * _CB + g * _L, _L)]
                      for j in range(_NF)]
                rb = []
                for gi, grp in enumerate(_GROUPS):
                    idx = xg[grp[0]]
                    for j in grp[1:]:
                        idx = idx * _DIMS[j] + xg[j]
                    rb.append((idx + _GOFFS[gi]) * (_HID // 2))

                @pl.loop(0, _L)
                def _node(n, rb=rb, g=g, buf=buf):
                    pre = [_splat(r, n) + iota for r in rb]
                    rowv = jnp.full((_L,), buf * _CB + g * _L + n, jnp.int32)

                    @plsc.parallel_loop(0, _HID // 2, _L, unroll=8)
                    def _cols(col, pre=pre, rowv=rowv):
                        acc = None
                        for p in pre:
                            w = plsc.bitcast(
                                plsc.load_gather(tbl_v, [p + col]),
                                jnp.bfloat16)
                            acc = w if acc is None else acc + w
                        acc_lo, acc_hi = plsc.unpack(
                            acc, format=plsc.PackFormat.INTERLEAVED,
                            preferred_element_type=jnp.float32)
                        plsc.store_scatter(
                            out_v, [rowv, iota + col], acc_lo)
                        plsc.store_scatter(
                            out_v, [rowv, iota + (col + _HID // 2)],
                            acc_hi)

            pltpu.make_async_copy(
                out_v.at[pl.ds(buf * _CB, _CB)],
                out.at[pl.ds(chunk_base(kk), _CB)],
                out_sems[buf]).start()

        @pl.loop(0, (kmax + 1) // 2)
        def _pair(kk2):
            kk = kk2 * 2
            do_chunk(kk, 0)

            @pl.when(kk + 1 < kmax)
            def _():
                do_chunk(kk + 1, 1)

        for buf in range(2):
            pltpu.make_async_copy(
                out_v.at[pl.ds(buf * _CB, _CB)], out.at[pl.ds(0, _CB)],
                out_sems[buf]).wait()

    return sc_kernel


def kernel(x, tables):
    tflat = _build_table(tables)
    x_f = x.T.reshape(-1)
    return _make(x.shape[0])(x_f, tflat)


# speedup vs baseline: 1.9715x; 1.1860x over previous
import functools

import jax
import jax.numpy as jnp
from jax import lax
from jax.experimental import pallas as pl
from jax.experimental.pallas import tpu as pltpu
from jax.experimental.pallas import tpu_sc as plsc

_DIMS = (119, 5, 12, 12, 10, 6, 6, 2, 2)
_HID = 256
_GROUPS = ((0,), (1, 2), (3, 4), (5, 6, 7, 8))
_L = 16
_NC = 2
_NS = 16
_NW = _NC * _NS
_CB = 128
_NF = len(_DIMS)


def _group_layout():
    sizes = []
    for grp in _GROUPS:
        r = 1
        for j in grp:
            r *= _DIMS[j]
        sizes.append(r)
    offs, acc = [], 0
    for s in sizes:
        offs.append(acc)
        acc += s
    return tuple(sizes), tuple(offs), acc


_GSIZES, _GOFFS, _TROWS = _group_layout()


def _build_table(tables):
    parts = []
    for grp in _GROUPS:
        t = tables[grp[0]]
        for j in grp[1:]:
            t = (t[:, None, :] + tables[j][None, :, :]).reshape(-1, _HID)
        parts.append(t)
    tb = jnp.concatenate(parts, 0).astype(jnp.bfloat16)
    lo = lax.bitcast_convert_type(tb[:, :_HID // 2], jnp.uint16)
    hi = lax.bitcast_convert_type(tb[:, _HID // 2:], jnp.uint16)
    packed = (hi.astype(jnp.uint32) << 16) | lo.astype(jnp.uint32)
    return lax.bitcast_convert_type(packed, jnp.int32).reshape(-1)


def _splat(vec, lane):
    idx = jnp.full((_L, 1), lane, jnp.int32)
    return lax.gather(
        vec, idx,
        dimension_numbers=lax.GatherDimensionNumbers(
            offset_dims=(), collapsed_slice_dims=(0,), start_index_map=(0,)),
        slice_sizes=(1,),
        mode=lax.GatherScatterMode.PROMISE_IN_BOUNDS)


@functools.lru_cache(maxsize=None)
def _make(n_nodes):
    n_chunks = -(-n_nodes // _CB)
    kmax = -(-n_chunks // _NW)
    last_base = n_nodes - _CB

    mesh = plsc.VectorSubcoreMesh(
        core_axis_name="c", subcore_axis_name="s",
        num_cores=_NC, num_subcores=_NS)

    @functools.partial(
        pl.kernel,
        out_type=jax.ShapeDtypeStruct((n_nodes, _HID), jnp.float32),
        mesh=mesh,
        scratch_types=[
            pltpu.VMEM((_TROWS * _HID // 2,), jnp.int32),
            pltpu.VMEM((2 * _NF * _CB,), jnp.int32),
            pltpu.VMEM((2 * _CB, _HID), jnp.float32),
            pltpu.SemaphoreType.DMA,
            pltpu.SemaphoreType.DMA,
            pltpu.SemaphoreType.DMA,
            pltpu.SemaphoreType.DMA,
        ],
        compiler_params=pltpu.CompilerParams(needs_layout_passes=False),
    )
    def sc_kernel(x_f, tflat, out, tbl_v, xv, out_v,
                  in_sem0, in_sem1, out_sem0, out_sem1):
        in_sems = (in_sem0, in_sem1)
        out_sems = (out_sem0, out_sem1)
        wid = lax.axis_index("s") * _NC + lax.axis_index("c")
        pltpu.sync_copy(tflat, tbl_v)
        iota = lax.broadcasted_iota(jnp.int32, (_L,), 0)

        def chunk_base(kk):
            cid = jnp.minimum(wid + _NW * kk, n_chunks - 1)
            return jnp.minimum(cid * _CB, last_base)

        def in_copies(kk, buf):
            base = chunk_base(kk)
            return [pltpu.make_async_copy(
                        x_f.at[pl.ds(j * n_nodes + base, _CB)],
                        xv.at[pl.ds((buf * _NF + j) * _CB, _CB)],
                        in_sems[buf])
                    for j in range(_NF)]

        def start_in(kk, buf):
            for c in in_copies(kk, buf):
                c.start()

        start_in(0, 0)

        def do_chunk(kk, buf):
            @pl.when(kk + 1 < kmax)
            def _():
                start_in(kk + 1, 1 - buf)

            for c in in_copies(kk, buf):
                c.wait()

            @pl.when(kk >= 2)
            def _():
                pltpu.make_async_copy(
                    out_v.at[pl.ds(buf * _CB, _CB)],
                    out.at[pl.ds(chunk_base(kk), _CB)],
                    out_sems[buf]).wait()

            for g in range(_CB // _L):
                xg = [xv[pl.ds((buf * _NF + j) * _CB + g * _L, _L)]
                      for j in range(_NF)]
                rb = []
                for gi, grp in enumerate(_GROUPS):
                    idx = xg[grp[0]]
                    for j in grp[1:]:
                        idx = idx * _DIMS[j] + xg[j]
                    rb.append((idx + _GOFFS[gi]) * (_HID // 2))

                @pl.loop(0, _L, unroll=2)
                def _node(n, rb=rb, g=g, buf=buf):
                    pre = [_splat(r, n) + iota for r in rb]
                    rowv = jnp.full((_L,), buf * _CB + g * _L + n, jnp.int32)

                    @plsc.parallel_loop(0, _HID // 2, _L, unroll=8)
                    def _cols(col, pre=pre, rowv=rowv):
                        acc = None
                        for p in pre:
                            w = plsc.bitcast(
                                plsc.load_gather(tbl_v, [p + col]),
                                jnp.bfloat16)
                            acc = w if acc is None else acc + w
                        acc_lo, acc_hi = plsc.unpack(
                            acc, format=plsc.PackFormat.INTERLEAVED,
                            preferred_element_type=jnp.float32)
                        plsc.store_scatter(
                            out_v, [rowv, iota + col], acc_lo)
                        plsc.store_scatter(
                            out_v, [rowv, iota + (col + _HID // 2)],
                            acc_hi)

            pltpu.make_async_copy(
                out_v.at[pl.ds(buf * _CB, _CB)],
                out.at[pl.ds(chunk_base(kk), _CB)],
                out_sems[buf]).start()

        @pl.loop(0, (kmax + 1) // 2)
        def _pair(kk2):
            kk = kk2 * 2
            do_chunk(kk, 0)

            @pl.when(kk + 1 < kmax)
            def _():
                do_chunk(kk + 1, 1)

        for buf in range(2):
            pltpu.make_async_copy(
                out_v.at[pl.ds(buf * _CB, _CB)], out.at[pl.ds(0, _CB)],
                out_sems[buf]).wait()

    return sc_kernel


def kernel(x, tables):
    tflat = _build_table(tables)
    x_f = x.T.reshape(-1)
    return _make(x.shape[0])(x_f, tflat)
